# baseline (speedup 1.0000x reference)
import jax
import jax.numpy as jnp
from jax.experimental import pallas as pl

B, N, S, K = 4, 16384, 1024, 32
C1, C2, CL = 64, 128, 256


def _fps_xla(pts, npoints):
    Bv, Nv, _ = pts.shape

    def body(i, carry):
        dists, idx, farthest = carry
        idx = idx.at[:, i].set(farthest)
        centroid = jnp.take_along_axis(pts, farthest[:, None, None], axis=1)
        d = jnp.sum((pts - centroid) ** 2, axis=-1)
        dists = jnp.minimum(dists, d)
        farthest = jnp.argmax(dists, axis=-1).astype(jnp.int32)
        return (dists, idx, farthest)

    dists0 = jnp.full((Bv, Nv), 1e10, dtype=jnp.float32)
    idx0 = jnp.zeros((Bv, npoints), dtype=jnp.int32)
    far0 = jnp.zeros((Bv,), dtype=jnp.int32)
    _, idx, _ = jax.lax.fori_loop(0, npoints, body, (dists0, idx0, far0))
    return idx


def kernel(xyz, W1, b1, g1, be1, W2, b2, g2, be2, Wl, bl):
    pts = jnp.transpose(xyz, (0, 2, 1))
    fps_idx = _fps_xla(pts, S)
    sampled = jnp.take_along_axis(pts, fps_idx[:, :, None], axis=1)
    d2 = (jnp.sum(sampled ** 2, axis=-1)[:, :, None]
          - 2.0 * jnp.einsum('bsc,bnc->bsn', sampled, pts)
          + jnp.sum(pts ** 2, axis=-1)[:, None, :])
    _, knn_idx = jax.lax.top_k(-d2, K)
    batch_ix = jnp.arange(B)[:, None, None]
    grouped = pts[batch_ix, knn_idx]
    grouped = grouped - sampled[:, :, None, :]
    x = jnp.transpose(grouped, (0, 3, 1, 2))

    def conv(x, W, b):
        return jnp.einsum('oc,bcsk->bosk', W, x) + b[None, :, None, None]

    def bn(x, g, be):
        mu = jnp.mean(x, axis=(0, 2, 3), keepdims=True)
        var = jnp.var(x, axis=(0, 2, 3), keepdims=True)
        return g[None, :, None, None] * (x - mu) / jnp.sqrt(var + 1e-5) + be[None, :, None, None]

    x = jax.nn.relu(bn(conv(x, W1, b1), g1, be1))
    x = jax.nn.relu(bn(conv(x, W2, b2), g2, be2))
    x = conv(x, Wl, bl)
    new_points = jnp.max(x, axis=-1)
    sampled_xyz = jnp.transpose(sampled, (0, 2, 1))
    return (sampled_xyz, new_points)


# speedup vs baseline: 1.7123x; 1.7123x over previous
import jax
import jax.numpy as jnp
from jax import lax
from jax.experimental import pallas as pl
from jax.experimental.pallas import tpu as pltpu

B, N, S, K = 4, 16384, 1024, 32
C1, C2, CL = 64, 128, 256


def _fps_body(xt_ref, sx_ref, sy_ref, sz_ref):
    X = xt_ref[0]
    Y = xt_ref[1]
    Z = xt_ref[2]
    iota_n = lax.broadcasted_iota(jnp.int32, (B, N), 1)
    dists0 = jnp.full((B, N), 1e10, dtype=jnp.float32)
    far0 = jnp.zeros((B, 1), dtype=jnp.int32)

    def body(i, carry):
        dists, far = carry
        sel = iota_n == far
        neg = jnp.float32(-jnp.inf)
        cx = jnp.max(jnp.where(sel, X, neg), axis=1, keepdims=True)
        cy = jnp.max(jnp.where(sel, Y, neg), axis=1, keepdims=True)
        cz = jnp.max(jnp.where(sel, Z, neg), axis=1, keepdims=True)
        sx_ref[pl.ds(i, 1), :] = cx.reshape(1, B)
        sy_ref[pl.ds(i, 1), :] = cy.reshape(1, B)
        sz_ref[pl.ds(i, 1), :] = cz.reshape(1, B)
        dx = X - cx
        dy = Y - cy
        dz = Z - cz
        d = dx * dx + dy * dy + dz * dz
        dists = jnp.minimum(dists, d)
        maxv = jnp.max(dists, axis=1, keepdims=True)
        far = jnp.min(jnp.where(dists == maxv, iota_n, N), axis=1, keepdims=True)
        return (dists, far.astype(jnp.int32))

    lax.fori_loop(0, S, body, (dists0, far0))


def _fps_pallas(xt):
    out = pl.pallas_call(
        _fps_body,
        out_shape=(
            jax.ShapeDtypeStruct((S, B), jnp.float32),
            jax.ShapeDtypeStruct((S, B), jnp.float32),
            jax.ShapeDtypeStruct((S, B), jnp.float32),
        ),
    )(xt)
    return out


def kernel(xyz, W1, b1, g1, be1, W2, b2, g2, be2, Wl, bl):
    pts = jnp.transpose(xyz, (0, 2, 1))
    xt = jnp.transpose(xyz, (1, 0, 2))
    sx, sy, sz = _fps_pallas(xt)
    sampled = jnp.stack([sx.T, sy.T, sz.T], axis=-1)
    d2 = (jnp.sum(sampled ** 2, axis=-1)[:, :, None]
          - 2.0 * jnp.einsum('bsc,bnc->bsn', sampled, pts)
          + jnp.sum(pts ** 2, axis=-1)[:, None, :])
    _, knn_idx = jax.lax.top_k(-d2, K)
    batch_ix = jnp.arange(B)[:, None, None]
    grouped = pts[batch_ix, knn_idx]
    grouped = grouped - sampled[:, :, None, :]
    x = jnp.transpose(grouped, (0, 3, 1, 2))

    def conv(x, W, b):
        return jnp.einsum('oc,bcsk->bosk', W, x) + b[None, :, None, None]

    def bn(x, g, be):
        mu = jnp.mean(x, axis=(0, 2, 3), keepdims=True)
        var = jnp.var(x, axis=(0, 2, 3), keepdims=True)
        return g[None, :, None, None] * (x - mu) / jnp.sqrt(var + 1e-5) + be[None, :, None, None]

    x = jax.nn.relu(bn(conv(x, W1, b1), g1, be1))
    x = jax.nn.relu(bn(conv(x, W2, b2), g2, be2))
    x = conv(x, Wl, bl)
    new_points = jnp.max(x, axis=-1)
    sampled_xyz = jnp.transpose(sampled, (0, 2, 1))
    return (sampled_xyz, new_points)


# speedup vs baseline: 9.6491x; 5.6353x over previous
import jax
import jax.numpy as jnp
from jax import lax
from jax.experimental import pallas as pl
from jax.experimental.pallas import tpu as pltpu

B, N, S, K = 4, 16384, 1024, 32
C1, C2, CL = 64, 128, 256


def _fps_body(xt_ref, sx_ref, sy_ref, sz_ref):
    X = xt_ref[0]
    Y = xt_ref[1]
    Z = xt_ref[2]
    iota_n = lax.broadcasted_iota(jnp.int32, (B, N), 1)
    dists0 = jnp.full((B, N), 1e10, dtype=jnp.float32)
    far0 = jnp.zeros((B, 1), dtype=jnp.int32)

    def body(i, carry):
        dists, far = carry
        sel = iota_n == far
        neg = jnp.float32(-jnp.inf)
        cx = jnp.max(jnp.where(sel, X, neg), axis=1, keepdims=True)
        cy = jnp.max(jnp.where(sel, Y, neg), axis=1, keepdims=True)
        cz = jnp.max(jnp.where(sel, Z, neg), axis=1, keepdims=True)
        sx_ref[pl.ds(i, 1), :] = cx.reshape(1, B)
        sy_ref[pl.ds(i, 1), :] = cy.reshape(1, B)
        sz_ref[pl.ds(i, 1), :] = cz.reshape(1, B)
        dx = X - cx
        dy = Y - cy
        dz = Z - cz
        d = dx * dx + dy * dy + dz * dz
        dists = jnp.minimum(dists, d)
        maxv = jnp.max(dists, axis=1, keepdims=True)
        far = jnp.min(jnp.where(dists == maxv, iota_n, N), axis=1, keepdims=True)
        return (dists, far.astype(jnp.int32))

    lax.fori_loop(0, S, body, (dists0, far0))


def _fps_pallas(xt):
    out = pl.pallas_call(
        _fps_body,
        out_shape=(
            jax.ShapeDtypeStruct((S, B), jnp.float32),
            jax.ShapeDtypeStruct((S, B), jnp.float32),
            jax.ShapeDtypeStruct((S, B), jnp.float32),
        ),
    )(xt)
    return out


def _cmpex(va, ia, vb, ib):
    swap = (vb < va) | ((vb == va) & (ib < ia))
    lo_v = jnp.where(swap, vb, va)
    lo_i = jnp.where(swap, ib, ia)
    hi_v = jnp.where(swap, va, vb)
    hi_i = jnp.where(swap, ia, ib)
    return lo_v, lo_i, hi_v, hi_i


def _flip1(x):
    L = x.shape[1]
    if L == 1:
        return x
    return jnp.concatenate([x[:, j:j + 1] for j in range(L - 1, -1, -1)], axis=1)


def _stage_flip(v, i, sz):
    A, L, sub, W = v.shape
    v5 = v.reshape(A * (L // sz), sz, sub, W)
    i5 = i.reshape(A * (L // sz), sz, sub, W)
    av, bv = v5[:, :sz // 2], _flip1(v5[:, sz // 2:])
    ai, bi = i5[:, :sz // 2], _flip1(i5[:, sz // 2:])
    lo_v, lo_i, hi_v, hi_i = _cmpex(av, ai, bv, bi)
    ov = jnp.concatenate([lo_v, _flip1(hi_v)], axis=1)
    oi = jnp.concatenate([lo_i, _flip1(hi_i)], axis=1)
    return ov.reshape(A, L, sub, W), oi.reshape(A, L, sub, W)


def _stage_dist(v, i, dist):
    A, L, sub, W = v.shape
    v5 = v.reshape(A * (L // (2 * dist)), 2, dist, sub, W)
    i5 = i.reshape(A * (L // (2 * dist)), 2, dist, sub, W)
    lo_v, lo_i, hi_v, hi_i = _cmpex(v5[:, 0], i5[:, 0], v5[:, 1], i5[:, 1])
    ov = jnp.concatenate([lo_v[:, None], hi_v[:, None]], axis=1)
    oi = jnp.concatenate([lo_i[:, None], hi_i[:, None]], axis=1)
    return ov.reshape(A, L, sub, W), oi.reshape(A, L, sub, W)


def _sort_k(v, i):
    L = v.shape[1]
    sz = 2
    while sz <= L:
        v, i = _stage_flip(v, i, sz)
        d = sz // 4
        while d >= 1:
            v, i = _stage_dist(v, i, d)
            d //= 2
        sz *= 2
    return v, i


def _merge_keep_lo(av, ai, bv, bi):
    L = av.shape[1]
    bv = _flip1(bv)
    bi = _flip1(bi)
    lo_v, lo_i, _, _ = _cmpex(av, ai, bv, bi)
    d = L // 2
    while d >= 1:
        lo_v, lo_i = _stage_dist(lo_v, lo_i, d)
        d //= 2
    return lo_v, lo_i


def _topk_body(pT_ref, sa_ref, oi_ref):
    pT = pT_ref[0]
    sa = sa_ref[0]
    ri = lax.broadcasted_iota(jnp.int32, (8, 128), 0)
    sb1 = jnp.where(ri < 3, sa, 0.0).astype(jnp.bfloat16)
    sb2 = jnp.where((ri >= 3) & (ri < 6), 1.0, 0.0).astype(jnp.bfloat16)
    e2 = lax.dot_general(pT, sb1, dimension_numbers=(((0,), (0,)), ((), ())),
                         preferred_element_type=jnp.float32)
    ppm = lax.dot_general(pT, sb2, dimension_numbers=(((0,), (0,)), ((), ())),
                          preferred_element_type=jnp.float32)
    ss = sa[3:4, :]
    d2 = (ss + e2) + ppm
    G = N // (K * 8)
    v = d2.reshape(G, K, 8, 128)
    gi = lax.broadcasted_iota(jnp.int32, (G, K, 8, 128), 0)
    ki = lax.broadcasted_iota(jnp.int32, (G, K, 8, 128), 1)
    si = lax.broadcasted_iota(jnp.int32, (G, K, 8, 128), 2)
    i = (gi * K + ki) * 8 + si

    v, i = _sort_k(v, i)

    while v.shape[0] > 1:
        A = v.shape[0]
        v5 = v.reshape(A // 2, 2, K, 8, 128)
        i5 = i.reshape(A // 2, 2, K, 8, 128)
        v, i = _merge_keep_lo(v5[:, 0], i5[:, 0], v5[:, 1], i5[:, 1])

    w = 4
    while w >= 1:
        av, bv = v[:, :, :w], v[:, :, w:2 * w]
        ai, bi = i[:, :, :w], i[:, :, w:2 * w]
        v, i = _merge_keep_lo(av, ai, bv, bi)
        w //= 2

    oi_ref[0, 0] = i.reshape(K, 128)


def _topk_pallas(p_augT, s_aug):
    return pl.pallas_call(
        _topk_body,
        grid=(B, S // 128),
        in_specs=[
            pl.BlockSpec((1, 8, N), lambda b, j: (b, 0, 0)),
            pl.BlockSpec((1, 8, 128), lambda b, j: (b, 0, j)),
        ],
        out_specs=pl.BlockSpec((1, 1, K, 128), lambda b, j: (b, j, 0, 0)),
        out_shape=jax.ShapeDtypeStruct((B, S // 128, K, 128), jnp.int32),
    )(p_augT, s_aug)


def kernel(xyz, W1, b1, g1, be1, W2, b2, g2, be2, Wl, bl):
    pts = jnp.transpose(xyz, (0, 2, 1))
    xt = jnp.transpose(xyz, (1, 0, 2))
    sx, sy, sz = _fps_pallas(xt)
    sampled = jnp.stack([sx.T, sy.T, sz.T], axis=-1)
    bf = jnp.bfloat16
    X, Y, Z = xyz[:, 0], xyz[:, 1], xyz[:, 2]
    pp = jnp.sum(pts ** 2, axis=-1)

    def _trunc_bf(v):
        return lax.bitcast_convert_type(
            lax.bitcast_convert_type(v, jnp.uint32) & jnp.uint32(0xFFFF0000),
            jnp.float32)

    pp_hi = _trunc_bf(pp)
    r1 = pp - pp_hi
    pp_mid = _trunc_bf(r1)
    pp_lo = r1 - pp_mid
    zn = jnp.zeros_like(X, dtype=bf)
    p_augT = jnp.stack([X.astype(bf), Y.astype(bf), Z.astype(bf),
                        pp_hi.astype(bf), pp_mid.astype(bf), pp_lo.astype(bf),
                        zn, zn], axis=1)
    sxT, syT, szT = sx.T, sy.T, sz.T
    ss = jnp.sum(sampled ** 2, axis=-1)
    zs = jnp.zeros_like(sxT)
    s_aug = jnp.stack([-2.0 * sxT, -2.0 * syT, -2.0 * szT,
                       ss, zs, zs, zs, zs], axis=1)
    oi = _topk_pallas(p_augT, s_aug)
    knn_idx = oi.transpose(0, 1, 3, 2).reshape(B, S, K)
    batch_ix = jnp.arange(B)[:, None, None]
    grouped = pts[batch_ix, knn_idx]
    grouped = grouped - sampled[:, :, None, :]
    x = jnp.transpose(grouped, (0, 3, 1, 2))

    def conv(x, W, b):
        return jnp.einsum('oc,bcsk->bosk', W, x) + b[None, :, None, None]

    def bn(x, g, be):
        mu = jnp.mean(x, axis=(0, 2, 3), keepdims=True)
        var = jnp.var(x, axis=(0, 2, 3), keepdims=True)
        return g[None, :, None, None] * (x - mu) / jnp.sqrt(var + 1e-5) + be[None, :, None, None]

    x = jax.nn.relu(bn(conv(x, W1, b1), g1, be1))
    x = jax.nn.relu(bn(conv(x, W2, b2), g2, be2))
    x = conv(x, Wl, bl)
    new_points = jnp.max(x, axis=-1)
    sampled_xyz = jnp.transpose(sampled, (0, 2, 1))
    return (sampled_xyz, new_points)
